# Initial kernel scaffold; baseline (speedup 1.0000x reference)
import jax
import jax.numpy as jnp
from jax.experimental import pallas as pl


def kernel(x, batch, W_base, W_embed, temperature, q):
    raise NotImplementedError("write your pallas kernel here")


# speedup vs baseline: 10.7410x; 10.7410x over previous
import jax
import jax.numpy as jnp
from jax.experimental import pallas as pl
from jax.experimental.pallas import tpu as pltpu

_K = 8
_BLK = 256


def _encode_kernel(x_ref, wb_ref, we_ref, xb_out, xe_out, xet_out):
    x = x_ref[:]
    xb_out[:] = jnp.dot(x, wb_ref[:], preferred_element_type=jnp.float32)
    xe = jnp.dot(x, we_ref[:], preferred_element_type=jnp.float32)
    xe_out[:] = xe
    xet_out[:] = xe.T


def _topk_kernel(t_ref, xeb_ref, xet_ref, q_ref, vals_out, idx_out):
    blk = xeb_ref.shape[0]
    n = xet_ref.shape[1]
    scale = jnp.exp(jnp.clip(t_ref[0, 0], -5.0, 5.0))
    xe_b = xeb_ref[:]
    xet = xet_ref[:]
    sq_row = jnp.sum(xe_b * xe_b, axis=1, keepdims=True)
    sq_col = jnp.sum(xet * xet, axis=0, keepdims=True)
    dots = jnp.dot(xe_b, xet, preferred_element_type=jnp.float32)
    d2 = jnp.maximum((sq_row + sq_col) - 2.0 * dots, 0.0)
    g = jnp.log(-jnp.log(q_ref[:] + 1e-8))
    work = g - scale * d2
    iota = jax.lax.broadcasted_iota(jnp.int32, (blk, n), 1)
    vals = []
    idxs = []
    for _ in range(_K):
        m = jnp.max(work, axis=1, keepdims=True)
        hit = work == m
        idx = jnp.min(jnp.where(hit, iota, n), axis=1, keepdims=True)
        vals.append(m)
        idxs.append(idx)
        work = jnp.where(iota == idx, -jnp.inf, work)
    vals_out[:] = jnp.concatenate(vals, axis=1)
    idx_out[:] = jnp.concatenate(idxs, axis=1)


def kernel(x, batch, W_base, W_embed, temperature, q):
    n, df = x.shape
    x_, xe, xet = pl.pallas_call(
        _encode_kernel,
        out_shape=(
            jax.ShapeDtypeStruct((n, df), jnp.float32),
            jax.ShapeDtypeStruct((n, df), jnp.float32),
            jax.ShapeDtypeStruct((df, n), jnp.float32),
        ),
    )(x, W_base, W_embed)

    nblk = n // _BLK
    t2d = temperature.reshape(1, 1)
    vals, idx = pl.pallas_call(
        _topk_kernel,
        grid=(nblk,),
        in_specs=[
            pl.BlockSpec((1, 1), lambda i: (0, 0)),
            pl.BlockSpec((_BLK, df), lambda i: (i, 0)),
            pl.BlockSpec((df, n), lambda i: (0, 0)),
            pl.BlockSpec((_BLK, n), lambda i: (i, 0)),
        ],
        out_specs=(
            pl.BlockSpec((_BLK, _K), lambda i: (i, 0)),
            pl.BlockSpec((_BLK, _K), lambda i: (i, 0)),
        ),
        out_shape=(
            jax.ShapeDtypeStruct((n, _K), jnp.float32),
            jax.ShapeDtypeStruct((n, _K), jnp.int32),
        ),
        compiler_params=pltpu.CompilerParams(
            dimension_semantics=("arbitrary",),
        ),
    )(t2d, xe, xet, q)

    rows = jnp.tile(jnp.arange(n, dtype=idx.dtype).reshape(1, n, 1), (1, 1, _K))
    edges = jnp.stack((idx.reshape(1, -1), rows.reshape(1, -1)), axis=-2)
    return (x_, xe, edges, vals)


# speedup vs baseline: 12.5824x; 1.1714x over previous
import jax
import jax.numpy as jnp
from jax.experimental import pallas as pl
from jax.experimental.pallas import tpu as pltpu

_K = 8
_BLK = 256


def _encode_kernel(x_ref, wb_ref, we_ref, xb_out, xe_out, xet_out):
    x = x_ref[:]
    xb_out[:] = jnp.dot(x, wb_ref[:], preferred_element_type=jnp.float32)
    xe = jnp.dot(x, we_ref[:], preferred_element_type=jnp.float32)
    xe_out[:] = xe
    xet_out[:] = xe.T


def _topk_kernel(t_ref, xeb_ref, xet_ref, q_ref, vals_out, idx_out):
    blk = xeb_ref.shape[0]
    n = xet_ref.shape[1]
    scale = jnp.exp(jnp.clip(t_ref[0, 0], -5.0, 5.0))
    xe_b = xeb_ref[:]
    xet = xet_ref[:]
    sq_row = jnp.sum(xe_b * xe_b, axis=1, keepdims=True)
    sq_col = jnp.sum(xet * xet, axis=0, keepdims=True)
    dots = jnp.dot(xe_b, xet, preferred_element_type=jnp.float32)
    d2 = jnp.maximum((sq_row + sq_col) - 2.0 * dots, 0.0)
    g = jnp.log(-jnp.log(q_ref[:] + 1e-8))
    work = g - scale * d2
    iota_f = jax.lax.broadcasted_iota(jnp.int32, (blk, n), 1).astype(jnp.float32)
    vals = []
    idxs = []
    for _ in range(_K):
        m = jnp.max(work, axis=1, keepdims=True)
        cand = jnp.where(work == m, iota_f, jnp.inf)
        idx = jnp.min(cand, axis=1, keepdims=True)
        vals.append(m)
        idxs.append(idx)
        work = jnp.where(cand == idx, -jnp.inf, work)
    vals_out[:] = jnp.concatenate(vals, axis=1)
    idx_out[:] = jnp.concatenate(idxs, axis=1).astype(jnp.int32)


def kernel(x, batch, W_base, W_embed, temperature, q):
    n, df = x.shape
    x_, xe, xet = pl.pallas_call(
        _encode_kernel,
        out_shape=(
            jax.ShapeDtypeStruct((n, df), jnp.float32),
            jax.ShapeDtypeStruct((n, df), jnp.float32),
            jax.ShapeDtypeStruct((df, n), jnp.float32),
        ),
    )(x, W_base, W_embed)

    nblk = n // _BLK
    t2d = temperature.reshape(1, 1)
    vals, idx = pl.pallas_call(
        _topk_kernel,
        grid=(nblk,),
        in_specs=[
            pl.BlockSpec((1, 1), lambda i: (0, 0)),
            pl.BlockSpec((_BLK, df), lambda i: (i, 0)),
            pl.BlockSpec((df, n), lambda i: (0, 0)),
            pl.BlockSpec((_BLK, n), lambda i: (i, 0)),
        ],
        out_specs=(
            pl.BlockSpec((_BLK, _K), lambda i: (i, 0)),
            pl.BlockSpec((_BLK, _K), lambda i: (i, 0)),
        ),
        out_shape=(
            jax.ShapeDtypeStruct((n, _K), jnp.float32),
            jax.ShapeDtypeStruct((n, _K), jnp.int32),
        ),
        compiler_params=pltpu.CompilerParams(
            dimension_semantics=("arbitrary",),
        ),
    )(t2d, xe, xet, q)

    rows = jnp.tile(jnp.arange(n, dtype=idx.dtype).reshape(1, n, 1), (1, 1, _K))
    edges = jnp.stack((idx.reshape(1, -1), rows.reshape(1, -1)), axis=-2)
    return (x_, xe, edges, vals)


# speedup vs baseline: 12.5904x; 1.0006x over previous
import jax
import jax.numpy as jnp
from jax.experimental import pallas as pl
from jax.experimental.pallas import tpu as pltpu

_K = 8
_BLK = 256


def _encode_kernel(x_ref, wb_ref, we_ref, xb_out, xe_out, xet_out):
    x = x_ref[:]
    xb_out[:] = jnp.dot(x, wb_ref[:], preferred_element_type=jnp.float32)
    xe = jnp.dot(x, we_ref[:], preferred_element_type=jnp.float32)
    xe_out[:] = xe
    xet_out[:] = xe.T


def _topk_kernel(t_ref, xeb_ref, xet_ref, q_ref, vals_out, idx_out):
    blk = xeb_ref.shape[0]
    n = xet_ref.shape[1]
    scale = jnp.exp(jnp.clip(t_ref[0, 0], -5.0, 5.0))
    xe_b = xeb_ref[:]
    xet = xet_ref[:]
    sq_row = jnp.sum(xe_b * xe_b, axis=1, keepdims=True)
    sq_col = jnp.sum(xet * xet, axis=0, keepdims=True)
    dots = jnp.dot(xe_b, xet, preferred_element_type=jnp.float32)
    d2 = jnp.maximum((sq_row + sq_col) - 2.0 * dots, 0.0)
    g = jnp.log(-jnp.log(q_ref[:] + 1e-8))
    work = g - scale * d2
    iota_f = jax.lax.broadcasted_iota(jnp.int32, (blk, n), 1).astype(jnp.float32)
    vals = []
    idxs = []
    for _ in range(_K):
        m = jnp.max(work, axis=1, keepdims=True)
        cand = jnp.where(work == m, iota_f, jnp.inf)
        idx = jnp.min(cand, axis=1, keepdims=True)
        vals.append(m)
        idxs.append(idx)
        work = jnp.where(cand == idx, -jnp.inf, work)
    vals_out[:] = jnp.concatenate(vals, axis=1)
    idx_out[:] = jnp.concatenate(idxs, axis=1).astype(jnp.int32)


def kernel(x, batch, W_base, W_embed, temperature, q):
    n, df = x.shape
    x_, xe, xet = pl.pallas_call(
        _encode_kernel,
        out_shape=(
            jax.ShapeDtypeStruct((n, df), jnp.float32),
            jax.ShapeDtypeStruct((n, df), jnp.float32),
            jax.ShapeDtypeStruct((df, n), jnp.float32),
        ),
    )(x, W_base, W_embed)

    nblk = n // _BLK
    t2d = temperature.reshape(1, 1)
    vals, idx = pl.pallas_call(
        _topk_kernel,
        grid=(nblk,),
        in_specs=[
            pl.BlockSpec((1, 1), lambda i: (0, 0)),
            pl.BlockSpec((_BLK, df), lambda i: (i, 0)),
            pl.BlockSpec((df, n), lambda i: (0, 0)),
            pl.BlockSpec((_BLK, n), lambda i: (i, 0)),
        ],
        out_specs=(
            pl.BlockSpec((_BLK, _K), lambda i: (i, 0)),
            pl.BlockSpec((_BLK, _K), lambda i: (i, 0)),
        ),
        out_shape=(
            jax.ShapeDtypeStruct((n, _K), jnp.float32),
            jax.ShapeDtypeStruct((n, _K), jnp.int32),
        ),
        compiler_params=pltpu.CompilerParams(
            dimension_semantics=("parallel",),
        ),
    )(t2d, xe, xet, q)

    rows = jnp.tile(jnp.arange(n, dtype=idx.dtype).reshape(1, n, 1), (1, 1, _K))
    edges = jnp.stack((idx.reshape(1, -1), rows.reshape(1, -1)), axis=-2)
    return (x_, xe, edges, vals)


# speedup vs baseline: 13.0351x; 1.0353x over previous
import jax
import jax.numpy as jnp
from jax.experimental import pallas as pl
from jax.experimental.pallas import tpu as pltpu

_K = 8
_BLK = 256


def _encode_kernel(x_ref, wb_ref, we_ref, xb_out, xe_out, xet2_out, sqc_out):
    x = x_ref[:]
    xb_out[:] = jnp.dot(x, wb_ref[:], preferred_element_type=jnp.float32)
    xe = jnp.dot(x, we_ref[:], preferred_element_type=jnp.float32)
    xe_out[:] = xe
    xet = xe.T
    xet2_out[:] = 2.0 * xet
    sqc_out[:] = jnp.sum(xet * xet, axis=0, keepdims=True)


def _topk_kernel(t_ref, xeb_ref, xet2_ref, sqc_ref, q_ref, vals_out, idx_out):
    blk = xeb_ref.shape[0]
    n = xet2_ref.shape[1]
    scale = jnp.exp(jnp.clip(t_ref[0, 0], -5.0, 5.0))
    xe_b = xeb_ref[:]
    sq_row = jnp.sum(xe_b * xe_b, axis=1, keepdims=True)
    sq_col = sqc_ref[:]
    dots2 = jnp.dot(xe_b, xet2_ref[:], preferred_element_type=jnp.float32)
    d2 = jnp.maximum((sq_row + sq_col) - dots2, 0.0)
    g = jnp.log(-jnp.log(q_ref[:] + 1e-8))
    work = g - scale * d2
    iota_f = jax.lax.broadcasted_iota(jnp.int32, (blk, n), 1).astype(jnp.float32)
    vals = []
    idxs = []
    for _ in range(_K):
        m = jnp.max(work, axis=1, keepdims=True)
        cand = jnp.where(work == m, iota_f, jnp.inf)
        idx = jnp.min(cand, axis=1, keepdims=True)
        vals.append(m)
        idxs.append(idx)
        work = jnp.where(cand == idx, -jnp.inf, work)
    vals_out[:] = jnp.concatenate(vals, axis=1)
    idx_out[:] = jnp.concatenate(idxs, axis=1).astype(jnp.int32)


def kernel(x, batch, W_base, W_embed, temperature, q):
    n, df = x.shape
    x_, xe, xet2, sqc = pl.pallas_call(
        _encode_kernel,
        out_shape=(
            jax.ShapeDtypeStruct((n, df), jnp.float32),
            jax.ShapeDtypeStruct((n, df), jnp.float32),
            jax.ShapeDtypeStruct((df, n), jnp.float32),
            jax.ShapeDtypeStruct((1, n), jnp.float32),
        ),
    )(x, W_base, W_embed)

    nblk = n // _BLK
    t2d = temperature.reshape(1, 1)
    vals, idx = pl.pallas_call(
        _topk_kernel,
        grid=(nblk,),
        in_specs=[
            pl.BlockSpec((1, 1), lambda i: (0, 0)),
            pl.BlockSpec((_BLK, df), lambda i: (i, 0)),
            pl.BlockSpec((df, n), lambda i: (0, 0)),
            pl.BlockSpec((1, n), lambda i: (0, 0)),
            pl.BlockSpec((_BLK, n), lambda i: (i, 0)),
        ],
        out_specs=(
            pl.BlockSpec((_BLK, _K), lambda i: (i, 0)),
            pl.BlockSpec((_BLK, _K), lambda i: (i, 0)),
        ),
        out_shape=(
            jax.ShapeDtypeStruct((n, _K), jnp.float32),
            jax.ShapeDtypeStruct((n, _K), jnp.int32),
        ),
        compiler_params=pltpu.CompilerParams(
            dimension_semantics=("arbitrary",),
        ),
    )(t2d, xe, xet2, sqc, q)

    rows = jnp.tile(jnp.arange(n, dtype=idx.dtype).reshape(1, n, 1), (1, 1, _K))
    edges = jnp.stack((idx.reshape(1, -1), rows.reshape(1, -1)), axis=-2)
    return (x_, xe, edges, vals)
